# baseline (speedup 1.0000x reference)
import functools

import jax
import jax.numpy as jnp
from jax import lax
from jax.experimental import pallas as pl
from jax.experimental.pallas import tpu as pltpu
from jax.experimental.pallas import tpu_sc as plsc

D = 32
PACK = 128 // D
NC = 2
NS = 16
NW = NC * NS
LANE = 128
NBUF = 4


def _make_kernel(batch, hist):
    ntc = batch // LANE
    ntr = D // 8

    mesh = plsc.VectorSubcoreMesh(core_axis_name="c", subcore_axis_name="s")

    @functools.partial(
        pl.kernel,
        mesh=mesh,
        out_type=jax.ShapeDtypeStruct((hist, ntr, ntc, 8, LANE), jnp.float32),
        compiler_params=pltpu.CompilerParams(
            use_tc_tiling_on_sc=False, needs_layout_passes=False
        ),
        scratch_types=[
            pltpu.VMEM((hist, LANE), jnp.int32),
            pltpu.VMEM((NBUF, LANE), jnp.int32),
        ]
        + [pltpu.VMEM((LANE, LANE), jnp.float32) for _ in range(NBUF)]
        + [pltpu.VMEM((ntr, 8, LANE + 1), jnp.float32) for _ in range(NBUF)]
        + [
            pltpu.SemaphoreType.DMA((NBUF,)),
            pltpu.SemaphoreType.DMA((NBUF,)),
        ],
    )
    def k(table_hbm, xt_hbm, out_hbm, x_v, q_v, *bufs):
        rows_v = bufs[:NBUF]
        t_v = bufs[NBUF : 2 * NBUF]
        gsem, ssem = bufs[2 * NBUF], bufs[2 * NBUF + 1]
        wid = lax.axis_index("s") * NC + lax.axis_index("c")
        pltpu.sync_copy(xt_hbm.at[:, pl.ds(wid * LANE, LANE)], x_v)

        iota = lax.iota(jnp.int32, 16)

        def stage_q(j, b):
            for v in range(LANE // 16):
                q_v[b, pl.ds(v * 16, 16)] = lax.shift_right_logical(
                    x_v[j, pl.ds(v * 16, 16)], 2
                )

        def gather(j, b):
            return pltpu.make_async_copy(
                table_hbm.at[q_v.at[b]], rows_v[b], gsem.at[b]
            )

        def store(j, b):
            return pltpu.make_async_copy(
                t_v[b].at[:, :, pl.ds(0, LANE)], out_hbm.at[j, :, wid], ssem.at[b]
            )

        fidx = [
            (
                lax.shift_right_logical(iota + f0, 3),
                lax.bitwise_and(iota + f0, jnp.full((16,), 7, jnp.int32)),
            )
            for f0 in range(0, D, 16)
        ]

        def transpose(j, b):
            g = rows_v[b]
            t = t_v[b]

            @plsc.parallel_loop(0, LANE, step=16, unroll=2)
            def _(bc0):
                o2v = lax.bitwise_and(x_v[j, pl.ds(bc0, 16)], PACK - 1) * D
                for dd in range(16):
                    bc = bc0 + dd
                    o2 = o2v[dd]
                    bcv = jnp.full((16,), 1, jnp.int32) * bc
                    for i, (trv, frv) in enumerate(fidx):
                        colv = iota + (o2 + i * 16)
                        vals = plsc.load_gather(g, [bcv, colv])
                        plsc.store_scatter(t, [trv, frv, bcv], vals)

        for b in range(NBUF):
            stage_q(b, b)
            gather(b, b).start()

        @pl.loop(0, hist, step=NBUF)
        def _(j0):
            for b in range(NBUF):
                j = j0 + b
                gather(j, b).wait()

                @pl.when(j >= NBUF)
                def _():
                    store(j - NBUF, b).wait()

                transpose(j, b)
                store(j, b).start()

                @pl.when(j + NBUF < hist)
                def _():
                    stage_q(j + NBUF, b)
                    gather(j + NBUF, b).start()

        for b in range(NBUF):
            store(hist - NBUF + b, b).wait()

    return k


def kernel(X, position_table):
    batch, hist = X.shape
    xt = X.astype(jnp.int32).T
    pt = position_table.reshape(position_table.shape[0] // PACK, PACK * D)
    out5 = _make_kernel(batch, hist)(pt, xt)
    return out5.transpose(2, 4, 0, 1, 3).reshape(batch, hist, D)


# speedup vs baseline: 1.3136x; 1.3136x over previous
import functools

import jax
import jax.numpy as jnp
from jax import lax
from jax.experimental import pallas as pl
from jax.experimental.pallas import tpu as pltpu
from jax.experimental.pallas import tpu_sc as plsc

D = 32
NC = 2
NS = 16
NW = NC * NS
LANE = 128
NBUF = 4


def _make_kernel(batch, hist):
    ntc = batch // LANE
    ntr = D // 8

    mesh = plsc.VectorSubcoreMesh(core_axis_name="c", subcore_axis_name="s")

    @functools.partial(
        pl.kernel,
        mesh=mesh,
        out_type=jax.ShapeDtypeStruct((hist, ntr, ntc, 8, LANE), jnp.float32),
        compiler_params=pltpu.CompilerParams(
            use_tc_tiling_on_sc=False, needs_layout_passes=False
        ),
        scratch_types=[
            pltpu.VMEM((hist, LANE), jnp.int32),
        ]
        + [pltpu.VMEM((LANE, D), jnp.float32) for _ in range(NBUF)]
        + [pltpu.VMEM((ntr, 8, LANE + 1), jnp.float32) for _ in range(NBUF)]
        + [
            pltpu.SemaphoreType.DMA((NBUF,)),
            pltpu.SemaphoreType.DMA((NBUF,)),
        ],
    )
    def k(table_hbm, xt_hbm, out_hbm, idx_v, *bufs):
        rows_v = bufs[:NBUF]
        t_v = bufs[NBUF : 2 * NBUF]
        gsem, ssem = bufs[2 * NBUF], bufs[2 * NBUF + 1]
        wid = lax.axis_index("s") * NC + lax.axis_index("c")
        pltpu.sync_copy(xt_hbm.at[:, pl.ds(wid * LANE, LANE)], idx_v)

        def gather(j, b):
            return pltpu.make_async_copy(
                table_hbm.at[idx_v.at[j]], rows_v[b], gsem.at[b]
            )

        def store(j, b):
            return pltpu.make_async_copy(
                t_v[b].at[:, :, pl.ds(0, LANE)], out_hbm.at[j, :, wid], ssem.at[b]
            )

        iota = lax.iota(jnp.int32, 16)

        fidx = [
            (
                lax.shift_right_logical(iota + f0, 3),
                lax.bitwise_and(iota + f0, jnp.full((16,), 7, jnp.int32)),
            )
            for f0 in range(0, D, 16)
        ]

        def transpose(b):
            g = rows_v[b]
            t = t_v[b]

            @plsc.parallel_loop(0, LANE, step=4, unroll=2)
            def _(bc0):
                for d in range(4):
                    bc = bc0 + d
                    bcv = jnp.full((16,), 1, jnp.int32) * bc
                    for i, (trv, frv) in enumerate(fidx):
                        vals = g[bc, pl.ds(i * 16, 16)]
                        plsc.store_scatter(t, [trv, frv, bcv], vals)

        for b in range(NBUF):
            gather(b, b).start()

        @pl.loop(0, hist, step=NBUF)
        def _(j0):
            for b in range(NBUF):
                j = j0 + b
                gather(j, b).wait()

                @pl.when(j >= NBUF)
                def _():
                    store(j - NBUF, b).wait()

                transpose(b)
                store(j, b).start()

                @pl.when(j + NBUF < hist)
                def _():
                    gather(j + NBUF, b).start()

        for b in range(NBUF):
            store(hist - NBUF + b, b).wait()

    return k


def kernel(X, position_table):
    batch, hist = X.shape
    xt = X.astype(jnp.int32).T
    npos, d = position_table.shape
    pt = position_table.reshape(npos * d // 128, 128)
    pt = lax.optimization_barrier(pt)
    pt = pt.reshape(npos, d)
    out5 = _make_kernel(batch, hist)(pt, xt)
    return out5.transpose(2, 4, 0, 1, 3).reshape(batch, hist, D)
